# baseline (device time: 349504 ns/iter reference)
import jax
import jax.numpy as jnp
from jax import lax
from jax.experimental import pallas as pl
from jax.experimental.pallas import tpu as pltpu

N_DEV = 8


def kernel(x, w_mat, scale_x, scale_w):
    m_tot, k_per = x.shape
    _, n = w_mat.shape
    m_per = m_tot // N_DEV

    def body(x_ref, w_ref, sx_ref, sw_ref, out_ref,
             comm_ref, send_sems, recv_sems):
        my = lax.axis_index("i")
        left = (my - 1) % N_DEV
        right = (my + 1) % N_DEV

        barrier_sem = pltpu.get_barrier_semaphore()
        for nbr in (left, right):
            pl.semaphore_signal(
                barrier_sem, inc=1,
                device_id=(nbr,), device_id_type=pl.DeviceIdType.MESH,
            )
        pl.semaphore_wait(barrier_sem, 2)

        w_bf = w_ref[:, :].astype(jnp.bfloat16)

        def contrib(c):
            xc = x_ref[pl.ds(c * m_per, m_per), :].astype(jnp.bfloat16)
            return jax.lax.dot(xc, w_bf, preferred_element_type=jnp.float32)

        comm_ref[0, :, :] = contrib((my - 1) % N_DEV)

        for h in range(N_DEV - 1):
            send_slot = h % 2
            recv_slot = (h + 1) % 2
            rdma = pltpu.make_async_remote_copy(
                src_ref=comm_ref.at[send_slot],
                dst_ref=comm_ref.at[recv_slot],
                send_sem=send_sems.at[h],
                recv_sem=recv_sems.at[h],
                device_id=(right,),
                device_id_type=pl.DeviceIdType.MESH,
            )
            rdma.start()
            rdma.wait()

            c = (my - 2 - h) % N_DEV
            acc = comm_ref[recv_slot, :, :] + contrib(c)
            if h < N_DEV - 2:
                comm_ref[recv_slot, :, :] = acc
            else:
                scale = sx_ref[0] * sw_ref[0]
                out_ref[:, :] = jnp.maximum(acc * scale, 0.0)

    return pl.pallas_call(
        body,
        out_shape=jax.ShapeDtypeStruct((m_per, n), jnp.float32),
        in_specs=[
            pl.BlockSpec(memory_space=pltpu.VMEM),
            pl.BlockSpec(memory_space=pltpu.VMEM),
            pl.BlockSpec(memory_space=pltpu.SMEM),
            pl.BlockSpec(memory_space=pltpu.SMEM),
        ],
        out_specs=pl.BlockSpec(memory_space=pltpu.VMEM),
        scratch_shapes=[
            pltpu.VMEM((2, m_per, n), jnp.float32),
            pltpu.SemaphoreType.DMA((N_DEV - 1,)),
            pltpu.SemaphoreType.DMA((N_DEV - 1,)),
        ],
        compiler_params=pltpu.CompilerParams(collective_id=0),
    )(x, w_mat, scale_x, scale_w)


# device time: 112115 ns/iter; 3.1174x vs baseline; 3.1174x over previous
import jax
import jax.numpy as jnp
from jax import lax
from jax.experimental import pallas as pl
from jax.experimental.pallas import tpu as pltpu

N_DEV = 8


def kernel(x, w_mat, scale_x, scale_w):
    m_tot, k_per = x.shape
    _, n = w_mat.shape
    m_per = m_tot // N_DEV
    nh = n // 2

    def body(x_ref, w_ref, sx_ref, sw_ref, out_ref,
             comm_p, comm_m, send_p, recv_p, send_m, recv_m):
        my = lax.axis_index("i")
        left = (my - 1) % N_DEV
        right = (my + 1) % N_DEV

        wp_bf = w_ref[:, :nh].astype(jnp.bfloat16)
        wm_bf = w_ref[:, nh:].astype(jnp.bfloat16)

        def contrib(c, w_half):
            xc = x_ref[pl.ds(c * m_per, m_per), :].astype(jnp.bfloat16)
            return jax.lax.dot(xc, w_half, preferred_element_type=jnp.float32)

        comm_p[0, :, :] = contrib((my - 1) % N_DEV, wp_bf).astype(jnp.bfloat16)
        comm_m[0, :, :] = contrib((my + 1) % N_DEV, wm_bf).astype(jnp.bfloat16)

        barrier_sem = pltpu.get_barrier_semaphore()
        for nbr in (left, right):
            pl.semaphore_signal(
                barrier_sem, inc=1,
                device_id=(nbr,), device_id_type=pl.DeviceIdType.MESH,
            )
        pl.semaphore_wait(barrier_sem, 2)

        sends = []
        for h in range(N_DEV - 1):
            rdma_p = pltpu.make_async_remote_copy(
                src_ref=comm_p.at[h],
                dst_ref=comm_p.at[h + 1],
                send_sem=send_p.at[h],
                recv_sem=recv_p.at[h],
                device_id=(right,),
                device_id_type=pl.DeviceIdType.MESH,
            )
            rdma_m = pltpu.make_async_remote_copy(
                src_ref=comm_m.at[h],
                dst_ref=comm_m.at[h + 1],
                send_sem=send_m.at[h],
                recv_sem=recv_m.at[h],
                device_id=(left,),
                device_id_type=pl.DeviceIdType.MESH,
            )
            rdma_p.start()
            rdma_m.start()
            sends.append((rdma_p, rdma_m))

            cp = (my - 2 - h) % N_DEV
            cm = (my + 2 + h) % N_DEV
            ap = contrib(cp, wp_bf)
            am = contrib(cm, wm_bf)

            rdma_p.wait_recv()
            rdma_m.wait_recv()
            if h < N_DEV - 2:
                comm_p[h + 1, :, :] = (
                    comm_p[h + 1, :, :].astype(jnp.float32) + ap
                ).astype(jnp.bfloat16)
                comm_m[h + 1, :, :] = (
                    comm_m[h + 1, :, :].astype(jnp.float32) + am
                ).astype(jnp.bfloat16)
            else:
                scale = sx_ref[0] * sw_ref[0]
                acc_p = comm_p[h + 1, :, :].astype(jnp.float32) + ap
                acc_m = comm_m[h + 1, :, :].astype(jnp.float32) + am
                out_ref[:, :nh] = jnp.maximum(acc_p * scale, 0.0)
                out_ref[:, nh:] = jnp.maximum(acc_m * scale, 0.0)

        for rdma_p, rdma_m in sends:
            rdma_p.wait_send()
            rdma_m.wait_send()

    return pl.pallas_call(
        body,
        out_shape=jax.ShapeDtypeStruct((m_per, n), jnp.float32),
        in_specs=[
            pl.BlockSpec(memory_space=pltpu.VMEM),
            pl.BlockSpec(memory_space=pltpu.VMEM),
            pl.BlockSpec(memory_space=pltpu.SMEM),
            pl.BlockSpec(memory_space=pltpu.SMEM),
        ],
        out_specs=pl.BlockSpec(memory_space=pltpu.VMEM),
        scratch_shapes=[
            pltpu.VMEM((N_DEV, m_per, nh), jnp.bfloat16),
            pltpu.VMEM((N_DEV, m_per, nh), jnp.bfloat16),
            pltpu.SemaphoreType.DMA((N_DEV - 1,)),
            pltpu.SemaphoreType.DMA((N_DEV - 1,)),
            pltpu.SemaphoreType.DMA((N_DEV - 1,)),
            pltpu.SemaphoreType.DMA((N_DEV - 1,)),
        ],
        compiler_params=pltpu.CompilerParams(collective_id=0),
    )(x, w_mat, scale_x, scale_w)


# device time: 94672 ns/iter; 3.6917x vs baseline; 1.1842x over previous
import jax
import jax.numpy as jnp
from jax import lax
from jax.experimental import pallas as pl
from jax.experimental.pallas import tpu as pltpu

N_DEV = 8
N_SUB = 2


def kernel(x, w_mat, scale_x, scale_w):
    m_tot, k_per = x.shape
    _, n = w_mat.shape
    m_per = m_tot // N_DEV
    nq = n // (2 * N_SUB)

    def body(x_ref, w_ref, sx_ref, sw_ref, out_ref,
             comm_p, comm_m, send_p, recv_p, send_m, recv_m):
        my = lax.axis_index("i")
        left = (my - 1) % N_DEV
        right = (my + 1) % N_DEV

        def col0(ring, b):
            return (0 if ring == "p" else N_SUB * nq) + b * nq

        w_bf = {
            (ring, b): w_ref[:, col0(ring, b):col0(ring, b) + nq].astype(
                jnp.bfloat16)
            for ring in ("p", "m") for b in range(N_SUB)
        }

        def contrib(c, ring, b):
            xc = x_ref[pl.ds(c * m_per, m_per), :].astype(jnp.bfloat16)
            return jax.lax.dot(
                xc, w_bf[(ring, b)], preferred_element_type=jnp.float32)

        def buf(ring):
            return comm_p if ring == "p" else comm_m

        for b in range(N_SUB):
            comm_p[0, b] = contrib((my - 1) % N_DEV, "p", b).astype(jnp.bfloat16)
            comm_m[0, b] = contrib((my + 1) % N_DEV, "m", b).astype(jnp.bfloat16)

        barrier_sem = pltpu.get_barrier_semaphore()
        for nbr in (left, right):
            pl.semaphore_signal(
                barrier_sem, inc=1,
                device_id=(nbr,), device_id_type=pl.DeviceIdType.MESH,
            )
        pl.semaphore_wait(barrier_sem, 2)

        def mk(ring, h, b):
            ssem, rsem, tgt = (
                (send_p, recv_p, right) if ring == "p"
                else (send_m, recv_m, left))
            return pltpu.make_async_remote_copy(
                src_ref=buf(ring).at[h, b],
                dst_ref=buf(ring).at[h + 1, b],
                send_sem=ssem.at[h, b],
                recv_sem=rsem.at[h, b],
                device_id=(tgt,),
                device_id_type=pl.DeviceIdType.MESH,
            )

        order = [("p", 0), ("m", 0), ("p", 1), ("m", 1)]
        rd = {}
        for ring, b in order:
            rd[(ring, 0, b)] = mk(ring, 0, b)
            rd[(ring, 0, b)].start()

        for h in range(N_DEV - 1):
            c_of = {"p": (my - 2 - h) % N_DEV, "m": (my + 2 + h) % N_DEV}
            a = {(ring, b): contrib(c_of[ring], ring, b).astype(jnp.bfloat16)
                 for ring, b in order}

            for ring, b in order:
                rd[(ring, h, b)].wait_recv()
                if h < N_DEV - 2:
                    buf(ring)[h + 1, b] = buf(ring)[h + 1, b] + a[(ring, b)]
                    nxt = mk(ring, h + 1, b)
                    rd[(ring, h + 1, b)] = nxt
                    nxt.start()
                else:
                    scale = sx_ref[0] * sw_ref[0]
                    acc = (buf(ring)[h + 1, b].astype(jnp.float32)
                           + a[(ring, b)].astype(jnp.float32))
                    c0 = col0(ring, b)
                    out_ref[:, c0:c0 + nq] = jnp.maximum(acc * scale, 0.0)

        for r in rd.values():
            r.wait_send()

    return pl.pallas_call(
        body,
        out_shape=jax.ShapeDtypeStruct((m_per, n), jnp.float32),
        in_specs=[
            pl.BlockSpec(memory_space=pltpu.VMEM),
            pl.BlockSpec(memory_space=pltpu.VMEM),
            pl.BlockSpec(memory_space=pltpu.SMEM),
            pl.BlockSpec(memory_space=pltpu.SMEM),
        ],
        out_specs=pl.BlockSpec(memory_space=pltpu.VMEM),
        scratch_shapes=[
            pltpu.VMEM((N_DEV, N_SUB, m_per, nq), jnp.bfloat16),
            pltpu.VMEM((N_DEV, N_SUB, m_per, nq), jnp.bfloat16),
            pltpu.SemaphoreType.DMA((N_DEV - 1, N_SUB)),
            pltpu.SemaphoreType.DMA((N_DEV - 1, N_SUB)),
            pltpu.SemaphoreType.DMA((N_DEV - 1, N_SUB)),
            pltpu.SemaphoreType.DMA((N_DEV - 1, N_SUB)),
        ],
        compiler_params=pltpu.CompilerParams(collective_id=0),
    )(x, w_mat, scale_x, scale_w)


# device time: 94637 ns/iter; 3.6931x vs baseline; 1.0004x over previous
import jax
import jax.numpy as jnp
from jax import lax
from jax.experimental import pallas as pl
from jax.experimental.pallas import tpu as pltpu

N_DEV = 8
N_SUB = 2


def kernel(x, w_mat, scale_x, scale_w):
    m_tot, k_per = x.shape
    _, n = w_mat.shape
    m_per = m_tot // N_DEV
    nq = n // (2 * N_SUB)

    def body(x_ref, w_ref, sx_ref, sw_ref, out_ref,
             comm_p, comm_m, send_p, recv_p, send_m, recv_m):
        my = lax.axis_index("i")
        left = (my - 1) % N_DEV
        right = (my + 1) % N_DEV

        def col0(ring, b):
            return (0 if ring == "p" else N_SUB * nq) + b * nq

        nh = N_SUB * nq
        w_bf = {"p": w_ref[:, :nh].astype(jnp.bfloat16),
                "m": w_ref[:, nh:].astype(jnp.bfloat16)}

        def contribs(h):
            c_of = {"p": (my - 2 - h) % N_DEV, "m": (my + 2 + h) % N_DEV}
            out = {}
            for ring in ("p", "m"):
                xc = x_ref[pl.ds(c_of[ring] * m_per, m_per), :].astype(
                    jnp.bfloat16)
                full = jax.lax.dot(
                    xc, w_bf[ring], preferred_element_type=jnp.float32)
                for b in range(N_SUB):
                    out[(ring, b)] = full[:, b * nq:(b + 1) * nq].astype(
                        jnp.bfloat16)
            return out

        def buf(ring):
            return comm_p if ring == "p" else comm_m

        seed = {"p": (my - 1) % N_DEV, "m": (my + 1) % N_DEV}
        for ring in ("p", "m"):
            xc = x_ref[pl.ds(seed[ring] * m_per, m_per), :].astype(jnp.bfloat16)
            full = jax.lax.dot(
                xc, w_bf[ring], preferred_element_type=jnp.float32)
            for b in range(N_SUB):
                buf(ring)[0, b] = full[:, b * nq:(b + 1) * nq].astype(
                    jnp.bfloat16)

        barrier_sem = pltpu.get_barrier_semaphore()
        for nbr in (left, right):
            pl.semaphore_signal(
                barrier_sem, inc=1,
                device_id=(nbr,), device_id_type=pl.DeviceIdType.MESH,
            )
        pl.semaphore_wait(barrier_sem, 2)

        def mk(ring, h, b):
            ssem, rsem, tgt = (
                (send_p, recv_p, right) if ring == "p"
                else (send_m, recv_m, left))
            return pltpu.make_async_remote_copy(
                src_ref=buf(ring).at[h, b],
                dst_ref=buf(ring).at[h + 1, b],
                send_sem=ssem.at[h, b],
                recv_sem=rsem.at[h, b],
                device_id=(tgt,),
                device_id_type=pl.DeviceIdType.MESH,
            )

        order = [("p", 0), ("m", 0), ("p", 1), ("m", 1)]
        rd = {}
        for ring, b in order:
            rd[(ring, 0, b)] = mk(ring, 0, b)
            rd[(ring, 0, b)].start()

        a = contribs(0)

        for h in range(N_DEV - 1):
            a_next = None
            for idx, (ring, b) in enumerate(order):
                rd[(ring, h, b)].wait_recv()
                if h < N_DEV - 2:
                    buf(ring)[h + 1, b] = buf(ring)[h + 1, b] + a[(ring, b)]
                    nxt = mk(ring, h + 1, b)
                    rd[(ring, h + 1, b)] = nxt
                    nxt.start()
                else:
                    scale = sx_ref[0] * sw_ref[0]
                    acc = (buf(ring)[h + 1, b].astype(jnp.float32)
                           + a[(ring, b)].astype(jnp.float32))
                    c0 = col0(ring, b)
                    out_ref[:, c0:c0 + nq] = jnp.maximum(acc * scale, 0.0)
                if idx == 1 and h < N_DEV - 2:
                    a_next = contribs(h + 1)
            a = a_next

        for r in rd.values():
            r.wait_send()

    return pl.pallas_call(
        body,
        out_shape=jax.ShapeDtypeStruct((m_per, n), jnp.float32),
        in_specs=[
            pl.BlockSpec(memory_space=pltpu.VMEM),
            pl.BlockSpec(memory_space=pltpu.VMEM),
            pl.BlockSpec(memory_space=pltpu.SMEM),
            pl.BlockSpec(memory_space=pltpu.SMEM),
        ],
        out_specs=pl.BlockSpec(memory_space=pltpu.VMEM),
        scratch_shapes=[
            pltpu.VMEM((N_DEV, N_SUB, m_per, nq), jnp.bfloat16),
            pltpu.VMEM((N_DEV, N_SUB, m_per, nq), jnp.bfloat16),
            pltpu.SemaphoreType.DMA((N_DEV - 1, N_SUB)),
            pltpu.SemaphoreType.DMA((N_DEV - 1, N_SUB)),
            pltpu.SemaphoreType.DMA((N_DEV - 1, N_SUB)),
            pltpu.SemaphoreType.DMA((N_DEV - 1, N_SUB)),
        ],
        compiler_params=pltpu.CompilerParams(collective_id=0),
    )(x, w_mat, scale_x, scale_w)


# device time: 93149 ns/iter; 3.7521x vs baseline; 1.0160x over previous
import os

import jax
import jax.numpy as jnp
from jax import lax
from jax.experimental import pallas as pl
from jax.experimental.pallas import tpu as pltpu

N_DEV = 8
N_SUB = 2

_ABLATE = os.environ.get("ABLATE", "")


def kernel(x, w_mat, scale_x, scale_w):
    m_tot, k_per = x.shape
    _, n = w_mat.shape
    m_per = m_tot // N_DEV
    nq = n // (2 * N_SUB)

    def body(x_ref, w_ref, sx_ref, sw_ref, out_ref,
             comm_p, comm_m, send_p, recv_p, send_m, recv_m):
        my = lax.axis_index("i")
        left = (my - 1) % N_DEV
        right = (my + 1) % N_DEV

        def col0(ring, b):
            return (0 if ring == "p" else N_SUB * nq) + b * nq

        nh = N_SUB * nq
        w_bf = {"p": w_ref[:, :nh].astype(jnp.bfloat16),
                "m": w_ref[:, nh:].astype(jnp.bfloat16)}

        def contribs(h):
            c_of = {"p": (my - 2 - h) % N_DEV, "m": (my + 2 + h) % N_DEV}
            out = {}
            for ring in ("p", "m"):
                xc = x_ref[pl.ds(c_of[ring] * m_per, m_per), :].astype(
                    jnp.bfloat16)
                full = jax.lax.dot(
                    xc, w_bf[ring], preferred_element_type=jnp.float32)
                for b in range(N_SUB):
                    out[(ring, b)] = full[:, b * nq:(b + 1) * nq].astype(
                        jnp.bfloat16)
            return out

        def buf(ring):
            return comm_p if ring == "p" else comm_m

        if _ABLATE != "comm":
            seed = {"p": (my - 1) % N_DEV, "m": (my + 1) % N_DEV}
            for ring in ("p", "m"):
                xc = x_ref[pl.ds(seed[ring] * m_per, m_per), :].astype(
                    jnp.bfloat16)
                full = jax.lax.dot(
                    xc, w_bf[ring], preferred_element_type=jnp.float32)
                for b in range(N_SUB):
                    buf(ring)[0, b] = full[:, b * nq:(b + 1) * nq].astype(
                        jnp.bfloat16)

        barrier_sem = pltpu.get_barrier_semaphore()
        for nbr in (left, right):
            pl.semaphore_signal(
                barrier_sem, inc=1,
                device_id=(nbr,), device_id_type=pl.DeviceIdType.MESH,
            )
        pl.semaphore_wait(barrier_sem, 2)

        def mk(ring, h, b):
            ssem, rsem, tgt = (
                (send_p, recv_p, right) if ring == "p"
                else (send_m, recv_m, left))
            return pltpu.make_async_remote_copy(
                src_ref=buf(ring).at[h, b],
                dst_ref=buf(ring).at[h + 1, b],
                send_sem=ssem.at[h, b],
                recv_sem=rsem.at[h, b],
                device_id=(tgt,),
                device_id_type=pl.DeviceIdType.MESH,
            )

        order = [("p", 0), ("m", 0), ("p", 1), ("m", 1)]
        rd = {}

        if _ABLATE == "compute":
            a = contribs(0)
            for h in range(N_DEV - 1):
                for idx, (ring, b) in enumerate(order):
                    if h < N_DEV - 2:
                        buf(ring)[h + 1, b] = buf(ring)[h + 1, b] + a[(ring, b)]
                    else:
                        scale = sx_ref[0] * sw_ref[0]
                        acc = (buf(ring)[h + 1, b].astype(jnp.float32)
                               + a[(ring, b)].astype(jnp.float32))
                        c0 = col0(ring, b)
                        out_ref[:, c0:c0 + nq] = jnp.maximum(acc * scale, 0.0)
                    if idx == 1 and h < N_DEV - 2:
                        a = contribs(h + 1)
            return

        for ring, b in order:
            rd[(ring, 0, b)] = mk(ring, 0, b)
            rd[(ring, 0, b)].start()

        if _ABLATE != "comm":
            a = contribs(0)

        for h in range(N_DEV - 1):
            a_next = None
            for idx, (ring, b) in enumerate(order):
                rd[(ring, h, b)].wait_recv()
                if h < N_DEV - 2:
                    if _ABLATE != "comm":
                        buf(ring)[h + 1, b] = buf(ring)[h + 1, b] + a[(ring, b)]
                    nxt = mk(ring, h + 1, b)
                    rd[(ring, h + 1, b)] = nxt
                    nxt.start()
                elif _ABLATE != "comm":
                    scale = sx_ref[0] * sw_ref[0]
                    acc = (buf(ring)[h + 1, b].astype(jnp.float32)
                           + a[(ring, b)].astype(jnp.float32))
                    c0 = col0(ring, b)
                    out_ref[:, c0:c0 + nq] = jnp.maximum(acc * scale, 0.0)
                if idx == 1 and h < N_DEV - 2 and _ABLATE != "comm":
                    a_next = contribs(h + 1)
            a = a_next

        if _ABLATE == "comm":
            out_ref[:, :] = jnp.zeros((m_per, n), jnp.float32)

        for r in rd.values():
            r.wait_send()

    return pl.pallas_call(
        body,
        out_shape=jax.ShapeDtypeStruct((m_per, n), jnp.float32),
        in_specs=[
            pl.BlockSpec(memory_space=pltpu.VMEM),
            pl.BlockSpec(memory_space=pltpu.VMEM),
            pl.BlockSpec(memory_space=pltpu.SMEM),
            pl.BlockSpec(memory_space=pltpu.SMEM),
        ],
        out_specs=pl.BlockSpec(memory_space=pltpu.VMEM),
        scratch_shapes=[
            pltpu.VMEM((N_DEV, N_SUB, m_per, nq), jnp.bfloat16),
            pltpu.VMEM((N_DEV, N_SUB, m_per, nq), jnp.bfloat16),
            pltpu.SemaphoreType.DMA((N_DEV - 1, N_SUB)),
            pltpu.SemaphoreType.DMA((N_DEV - 1, N_SUB)),
            pltpu.SemaphoreType.DMA((N_DEV - 1, N_SUB)),
            pltpu.SemaphoreType.DMA((N_DEV - 1, N_SUB)),
        ],
        compiler_params=pltpu.CompilerParams(collective_id=0),
    )(x, w_mat, scale_x, scale_w)
